# Initial kernel scaffold; baseline (speedup 1.0000x reference)
import jax
import jax.numpy as jnp
from jax.experimental import pallas as pl


def kernel(x, edge_index, edge_weight, W1l, b1l, W1r, W2l, b2l, W2r, Wlin, blin):
    raise NotImplementedError("write your pallas kernel here")


# speedup vs baseline: 5.2235x; 5.2235x over previous
import functools

import jax
import jax.numpy as jnp
from jax import lax
from jax.experimental import pallas as pl
from jax.experimental.pallas import tpu as pltpu
from jax.experimental.pallas import tpu_sc as plsc

N = 10000
NPAD = 10240
E = 320000
D_IN = 128
H = 32
C = 2

NC = 2
NS = 16
NW = NC * NS
PER_W = E // NW
K = 80
NCHUNK = PER_W // K
CW = 16
RPT = NPAD // NS


def _segsum_kernel(with_counts):
    mesh = plsc.VectorSubcoreMesh(core_axis_name="c", subcore_axis_name="s")
    if with_counts:
        out_type = [jax.ShapeDtypeStruct((NC, NPAD, H), jnp.float32),
                    jax.ShapeDtypeStruct((NC, NPAD, CW), jnp.float32)]
    else:
        out_type = jax.ShapeDtypeStruct((NC, NPAD, H), jnp.float32)
    scratch = [
        pltpu.VMEM((K,), jnp.int32),
        pltpu.VMEM((K,), jnp.int32),
        pltpu.VMEM((K, H), jnp.float32),
        pltpu.VMEM((K,), jnp.float32),
        pltpu.VMEM((RPT, H), jnp.float32),
        pltpu.VMEM_SHARED((NPAD, H), jnp.float32),
        pltpu.SemaphoreType.DMA,
    ]
    if with_counts:
        scratch += [
            pltpu.VMEM((K, CW), jnp.float32),
            pltpu.VMEM((RPT, CW), jnp.float32),
            pltpu.VMEM_SHARED((NPAD, CW), jnp.float32),
        ]

    def body(p_hbm, src_hbm, dst_hbm, w_hbm, out_hbm, *rest):
        if with_counts:
            (cnt_hbm, idx_src, idx_dst, rows, wbuf, zbuf, acc, sem,
             ones, zcnt, cacc) = rest
        else:
            (idx_src, idx_dst, rows, wbuf, zbuf, acc, sem) = rest
        c = lax.axis_index("c")
        s = lax.axis_index("s")
        wid = c * NS + s

        @pl.loop(0, RPT)
        def _zero(i):
            for j in range(0, H, 16):
                zbuf[i, pl.ds(j, 16)] = jnp.zeros((16,), jnp.float32)
            if with_counts:
                zcnt[i, pl.ds(0, CW)] = jnp.zeros((CW,), jnp.float32)

        pltpu.sync_copy(zbuf, acc.at[pl.ds(s * RPT, RPT)])
        if with_counts:
            pltpu.sync_copy(zcnt, cacc.at[pl.ds(s * RPT, RPT)])

            @pl.loop(0, K)
            def _fill(i):
                ones[i, pl.ds(0, CW)] = jnp.ones((CW,), jnp.float32)

        plsc.subcore_barrier()

        base0 = wid * PER_W

        @pl.loop(0, NCHUNK)
        def _chunk(j):
            base = base0 + j * K
            pltpu.sync_copy(src_hbm.at[pl.ds(base, K)], idx_src)
            pltpu.sync_copy(dst_hbm.at[pl.ds(base, K)], idx_dst)
            pltpu.sync_copy(w_hbm.at[pl.ds(base, K)], wbuf)
            pltpu.async_copy(p_hbm.at[idx_src], rows, sem).wait()

            @pl.loop(0, K, step=16)
            def _scale(i):
                wv16 = wbuf[pl.ds(i, 16)]
                for e in range(16):
                    wv = wv16[e]
                    for jj in range(0, H, 16):
                        rows[i + e, pl.ds(jj, 16)] = (
                            rows[i + e, pl.ds(jj, 16)] * wv)

            pltpu.sync_copy(rows, acc.at[idx_dst], add=True)
            if with_counts:
                pltpu.sync_copy(ones, cacc.at[idx_dst], add=True)

        plsc.subcore_barrier()

        sl = pl.ds(s * RPT, RPT)
        pltpu.sync_copy(acc.at[sl], out_hbm.at[c, sl])
        if with_counts:
            pltpu.sync_copy(cacc.at[sl], cnt_hbm.at[c, sl])

    return pl.kernel(body, out_type=out_type, mesh=mesh,
                     scratch_types=scratch,
                     compiler_params=pltpu.CompilerParams(
                         use_tc_tiling_on_sc=False))


_segsum_counts = _segsum_kernel(True)
_segsum_plain = _segsum_kernel(False)


def _proj2_body(x_ref, wa_ref, wb_ref, pa_ref, pb_ref):
    xv = x_ref[...]
    pa_ref[...] = jnp.dot(xv, wa_ref[...], preferred_element_type=jnp.float32)
    pb_ref[...] = jnp.dot(xv, wb_ref[...], preferred_element_type=jnp.float32)


def _proj2(x, wa, wb):
    m = x.shape[0]
    h = wa.shape[1]
    return pl.pallas_call(
        _proj2_body,
        out_shape=[jax.ShapeDtypeStruct((m, h), jnp.float32)] * 2,
    )(x, wa, wb)


def _mid_body(accp_ref, cntp_ref, r_ref, b_ref, wl_ref, wr_ref,
              p2_ref, r2_ref):
    sacc = accp_ref[0] + accp_ref[1]
    cnt = cntp_ref[0, :, 0:1] + cntp_ref[1, :, 0:1]
    aggr = sacc / jnp.maximum(cnt, 1.0)
    out = aggr + b_ref[...][None, :] + r_ref[...]
    nrm = jnp.sqrt(jnp.sum(out * out, axis=-1, keepdims=True))
    h = jax.nn.relu(out / jnp.maximum(nrm, 1e-12))
    p2_ref[...] = jnp.dot(h, wl_ref[...], preferred_element_type=jnp.float32)
    r2_ref[...] = jnp.dot(h, wr_ref[...], preferred_element_type=jnp.float32)


def _final_body(accp_ref, cntp_ref, r_ref, b_ref, wlin_ref, blin_ref,
                o_ref):
    sacc = accp_ref[0] + accp_ref[1]
    cnt = cntp_ref[0, :, 0:1] + cntp_ref[1, :, 0:1]
    aggr = sacc / jnp.maximum(cnt, 1.0)
    out = aggr + b_ref[...][None, :] + r_ref[...]
    nrm = jnp.sqrt(jnp.sum(out * out, axis=-1, keepdims=True))
    h = jax.nn.relu(out / jnp.maximum(nrm, 1e-12))
    o_ref[...] = (jnp.dot(h, wlin_ref[...], preferred_element_type=jnp.float32)
                  + blin_ref[...][None, :])


def kernel(x, edge_index, edge_weight, W1l, b1l, W1r, W2l, b2l, W2r,
           Wlin, blin):
    src = edge_index[0]
    dst = edge_index[1]
    xp = jnp.pad(x, ((0, NPAD - N), (0, 0)))

    p1, r1 = _proj2(xp, W1l, W1r)
    acc1, cnt = _segsum_counts(p1, src, dst, edge_weight)

    p2, r2 = pl.pallas_call(
        _mid_body,
        out_shape=[jax.ShapeDtypeStruct((NPAD, H), jnp.float32)] * 2,
    )(acc1, cnt, r1, b1l, W2l, W2r)

    acc2 = _segsum_plain(p2, src, dst, edge_weight)

    out = pl.pallas_call(
        _final_body,
        out_shape=jax.ShapeDtypeStruct((NPAD, C), jnp.float32),
    )(acc2, cnt, r2, b2l, Wlin, blin)
    return out[:N]


# speedup vs baseline: 12.1060x; 2.3176x over previous
import functools

import jax
import jax.numpy as jnp
from jax import lax
from jax.experimental import pallas as pl
from jax.experimental.pallas import tpu as pltpu
from jax.experimental.pallas import tpu_sc as plsc

N = 10000
NPAD = 10240
E = 320000
D_IN = 128
H = 32
C = 2

NC = 2
NS = 16
NW = NC * NS
PER_W = E // NW
K = 80
NCHUNK = PER_W // K
CW = 16
RPT = NPAD // NS


def _segsum_kernel(with_counts):
    mesh = plsc.VectorSubcoreMesh(core_axis_name="c", subcore_axis_name="s")
    if with_counts:
        out_type = [jax.ShapeDtypeStruct((NC, NPAD, H), jnp.float32),
                    jax.ShapeDtypeStruct((NC, NPAD, CW), jnp.float32)]
    else:
        out_type = jax.ShapeDtypeStruct((NC, NPAD, H), jnp.float32)
    scratch = [
        pltpu.VMEM((NCHUNK, K), jnp.int32),
        pltpu.VMEM((NCHUNK, K), jnp.int32),
        pltpu.VMEM((NCHUNK, K), jnp.float32),
        pltpu.VMEM((K, H), jnp.float32),
        pltpu.VMEM((K, H), jnp.float32),
        pltpu.VMEM((RPT, H), jnp.float32),
        pltpu.VMEM_SHARED((NPAD, H), jnp.float32),
        pltpu.SemaphoreType.DMA,
        pltpu.SemaphoreType.DMA,
    ]
    if with_counts:
        scratch += [
            pltpu.VMEM((K, CW), jnp.float32),
            pltpu.VMEM((RPT, CW), jnp.float32),
            pltpu.VMEM_SHARED((NPAD, CW), jnp.float32),
        ]

    def body(p_hbm, src_hbm, dst_hbm, w_hbm, out_hbm, *rest):
        if with_counts:
            (cnt_hbm, srcb, dstb, wb, rowsa, rowsb, zbuf, acc, sema, semb,
             ones, zcnt, cacc) = rest
        else:
            (srcb, dstb, wb, rowsa, rowsb, zbuf, acc, sema, semb) = rest
        c = lax.axis_index("c")
        s = lax.axis_index("s")
        wid = c * NS + s

        pltpu.async_copy(src_hbm.at[wid], srcb, sema)
        pltpu.async_copy(dst_hbm.at[wid], dstb, sema)
        pltpu.async_copy(w_hbm.at[wid], wb, sema)

        @pl.loop(0, RPT)
        def _zero(i):
            for j in range(0, H, 16):
                zbuf[i, pl.ds(j, 16)] = jnp.zeros((16,), jnp.float32)
            if with_counts:
                zcnt[i, pl.ds(0, CW)] = jnp.zeros((CW,), jnp.float32)

        pltpu.sync_copy(zbuf, acc.at[pl.ds(s * RPT, RPT)])
        if with_counts:
            pltpu.sync_copy(zcnt, cacc.at[pl.ds(s * RPT, RPT)])

            @pl.loop(0, K)
            def _fill(i):
                ones[i, pl.ds(0, CW)] = jnp.ones((CW,), jnp.float32)

        pltpu.make_async_copy(src_hbm.at[wid], srcb, sema).wait()
        pltpu.make_async_copy(dst_hbm.at[wid], dstb, sema).wait()
        pltpu.make_async_copy(w_hbm.at[wid], wb, sema).wait()
        plsc.subcore_barrier()

        def start_gather(j, buf, sem):
            pltpu.async_copy(p_hbm.at[srcb.at[j]], buf, sem)

        def wait_gather(buf, sem):
            pltpu.make_async_copy(p_hbm.at[srcb.at[0]], buf, sem).wait()

        def scale(buf, j):
            @pl.loop(0, K, step=16)
            def _scale(i):
                wv16 = wb[j, pl.ds(i, 16)]
                for e in range(16):
                    wv = wv16[e]
                    for jj in range(0, H, 16):
                        buf[i + e, pl.ds(jj, 16)] = (
                            buf[i + e, pl.ds(jj, 16)] * wv)

        def scatter(buf, j):
            pltpu.sync_copy(buf, acc.at[dstb.at[j]], add=True)
            if with_counts:
                pltpu.sync_copy(ones, cacc.at[dstb.at[j]], add=True)

        start_gather(0, rowsa, sema)

        @pl.loop(0, (NCHUNK - 1) // 2)
        def _pair(t):
            j0 = 2 * t
            wait_gather(rowsa, sema)
            start_gather(j0 + 1, rowsb, semb)
            scale(rowsa, j0)
            scatter(rowsa, j0)
            wait_gather(rowsb, semb)
            start_gather(j0 + 2, rowsa, sema)
            scale(rowsb, j0 + 1)
            scatter(rowsb, j0 + 1)

        wait_gather(rowsa, sema)
        scale(rowsa, NCHUNK - 1)
        scatter(rowsa, NCHUNK - 1)

        plsc.subcore_barrier()

        sl = pl.ds(s * RPT, RPT)
        pltpu.sync_copy(acc.at[sl], out_hbm.at[c, sl])
        if with_counts:
            pltpu.sync_copy(cacc.at[sl], cnt_hbm.at[c, sl])

    return pl.kernel(body, out_type=out_type, mesh=mesh,
                     scratch_types=scratch,
                     compiler_params=pltpu.CompilerParams(
                         use_tc_tiling_on_sc=False))


_segsum_counts = _segsum_kernel(True)
_segsum_plain = _segsum_kernel(False)


def _proj2_body(x_ref, wa_ref, wb_ref, pa_ref, pb_ref):
    xv = x_ref[...]
    pa_ref[...] = jnp.dot(xv, wa_ref[...], preferred_element_type=jnp.float32)
    pb_ref[...] = jnp.dot(xv, wb_ref[...], preferred_element_type=jnp.float32)


def _proj2(x, wa, wb):
    m = x.shape[0]
    h = wa.shape[1]
    return pl.pallas_call(
        _proj2_body,
        out_shape=[jax.ShapeDtypeStruct((m, h), jnp.float32)] * 2,
    )(x, wa, wb)


def _mid_body(accp_ref, cntp_ref, r_ref, b_ref, wl_ref, wr_ref,
              p2_ref, r2_ref):
    sacc = accp_ref[0] + accp_ref[1]
    cnt = cntp_ref[0, :, 0:1] + cntp_ref[1, :, 0:1]
    aggr = sacc / jnp.maximum(cnt, 1.0)
    out = aggr + b_ref[...][None, :] + r_ref[...]
    nrm = jnp.sqrt(jnp.sum(out * out, axis=-1, keepdims=True))
    h = jax.nn.relu(out / jnp.maximum(nrm, 1e-12))
    p2_ref[...] = jnp.dot(h, wl_ref[...], preferred_element_type=jnp.float32)
    r2_ref[...] = jnp.dot(h, wr_ref[...], preferred_element_type=jnp.float32)


def _final_body(accp_ref, cntp_ref, r_ref, b_ref, wlin_ref, blin_ref,
                o_ref):
    sacc = accp_ref[0] + accp_ref[1]
    cnt = cntp_ref[0, :, 0:1] + cntp_ref[1, :, 0:1]
    aggr = sacc / jnp.maximum(cnt, 1.0)
    out = aggr + b_ref[...][None, :] + r_ref[...]
    nrm = jnp.sqrt(jnp.sum(out * out, axis=-1, keepdims=True))
    h = jax.nn.relu(out / jnp.maximum(nrm, 1e-12))
    o_ref[...] = (jnp.dot(h, wlin_ref[...], preferred_element_type=jnp.float32)
                  + blin_ref[...][None, :])


def kernel(x, edge_index, edge_weight, W1l, b1l, W1r, W2l, b2l, W2r,
           Wlin, blin):
    src = edge_index[0].reshape(NW, NCHUNK, K)
    dst = edge_index[1].reshape(NW, NCHUNK, K)
    w3 = edge_weight.reshape(NW, NCHUNK, K)
    xp = jnp.pad(x, ((0, NPAD - N), (0, 0)))

    p1, r1 = _proj2(xp, W1l, W1r)
    acc1, cnt = _segsum_counts(p1, src, dst, w3)

    p2, r2 = pl.pallas_call(
        _mid_body,
        out_shape=[jax.ShapeDtypeStruct((NPAD, H), jnp.float32)] * 2,
    )(acc1, cnt, r1, b1l, W2l, W2r)

    acc2 = _segsum_plain(p2, src, dst, w3)

    out = pl.pallas_call(
        _final_body,
        out_shape=jax.ShapeDtypeStruct((NPAD, C), jnp.float32),
    )(acc2, cnt, r2, b2l, Wlin, blin)
    return out[:N]


# speedup vs baseline: 19.0803x; 1.5761x over previous
import functools

import jax
import jax.numpy as jnp
from jax import lax
from jax.experimental import pallas as pl
from jax.experimental.pallas import tpu as pltpu
from jax.experimental.pallas import tpu_sc as plsc

N = 10000
NPAD = 10240
E = 320000
D_IN = 128
H = 32
C = 2

NC = 2
NS = 16
NW = NC * NS
PER_W = E // NW
K = 80
NCHUNK = PER_W // K
CW = 16
RPT = NPAD // NS


def _segsum_kernel(with_counts):
    mesh = plsc.VectorSubcoreMesh(core_axis_name="c", subcore_axis_name="s")
    if with_counts:
        out_type = [jax.ShapeDtypeStruct((NC, NPAD, H), jnp.float32),
                    jax.ShapeDtypeStruct((NC, NPAD, CW), jnp.float32)]
    else:
        out_type = jax.ShapeDtypeStruct((NC, NPAD, H), jnp.float32)
    NBUF = 5
    scratch = (
        [pltpu.VMEM((NCHUNK, K), jnp.int32),
         pltpu.VMEM((NCHUNK, K), jnp.int32),
         pltpu.VMEM((NCHUNK, K), jnp.float32)]
        + [pltpu.VMEM((K, H), jnp.float32)] * NBUF
        + [pltpu.VMEM((RPT, H), jnp.float32),
           pltpu.VMEM_SHARED((NPAD, H), jnp.float32)]
        + [pltpu.SemaphoreType.DMA] * (2 * NBUF)
    )
    if with_counts:
        scratch += [
            pltpu.VMEM((K, CW), jnp.float32),
            pltpu.VMEM((RPT, CW), jnp.float32),
            pltpu.VMEM_SHARED((NPAD, CW), jnp.float32),
            pltpu.SemaphoreType.DMA,
        ]

    def body(p_hbm, src_hbm, dst_hbm, w_hbm, out_hbm, *rest):
        rest = list(rest)
        cnt_hbm = rest.pop(0) if with_counts else None
        srcb, dstb, wb = rest[0:3]
        bufs = rest[3:3 + NBUF]
        zbuf = rest[3 + NBUF]
        acc = rest[4 + NBUF]
        gsem = rest[5 + NBUF:5 + 2 * NBUF]
        ssem = rest[5 + 2 * NBUF:5 + 3 * NBUF]
        if with_counts:
            ones, zcnt, cacc, csem = rest[5 + 3 * NBUF:]
        c = lax.axis_index("c")
        s = lax.axis_index("s")
        wid = c * NS + s

        pltpu.async_copy(src_hbm.at[wid], srcb, gsem[0])
        pltpu.async_copy(dst_hbm.at[wid], dstb, gsem[0])
        pltpu.async_copy(w_hbm.at[wid], wb, gsem[0])

        @pl.loop(0, RPT)
        def _zero(i):
            for j in range(0, H, 16):
                zbuf[i, pl.ds(j, 16)] = jnp.zeros((16,), jnp.float32)
            if with_counts:
                zcnt[i, pl.ds(0, CW)] = jnp.zeros((CW,), jnp.float32)

        pltpu.sync_copy(zbuf, acc.at[pl.ds(s * RPT, RPT)])
        if with_counts:
            pltpu.sync_copy(zcnt, cacc.at[pl.ds(s * RPT, RPT)])

            @pl.loop(0, K)
            def _fill(i):
                ones[i, pl.ds(0, CW)] = jnp.ones((CW,), jnp.float32)

        pltpu.make_async_copy(src_hbm.at[wid], srcb, gsem[0]).wait()
        pltpu.make_async_copy(dst_hbm.at[wid], dstb, gsem[0]).wait()
        pltpu.make_async_copy(w_hbm.at[wid], wb, gsem[0]).wait()
        plsc.subcore_barrier()

        def start_gather(j, b):
            pltpu.async_copy(p_hbm.at[srcb.at[j]], bufs[b], gsem[b])

        def wait_gather(b):
            pltpu.make_async_copy(p_hbm.at[srcb.at[0]], bufs[b],
                                  gsem[b]).wait()

        def wait_scatter(b):
            pltpu.make_async_copy(bufs[b], acc.at[dstb.at[0]],
                                  ssem[b]).wait()

        def scale(buf, j):
            @pl.loop(0, K, step=16)
            def _scale(i):
                wv16 = wb[j, pl.ds(i, 16)]
                for e in range(16):
                    wv = wv16[e]
                    for jj in range(0, H, 16):
                        buf[i + e, pl.ds(jj, 16)] = (
                            buf[i + e, pl.ds(jj, 16)] * wv)

        def step(j, b, first=False, last=False):
            wait_gather(b)
            scale(bufs[b], j)
            pltpu.async_copy(bufs[b], acc.at[dstb.at[j]], ssem[b], add=True)
            if with_counts:
                if not first:
                    pltpu.make_async_copy(ones, cacc.at[dstb.at[0]],
                                          csem).wait()
                pltpu.async_copy(ones, cacc.at[dstb.at[j]], csem, add=True)
            if not last:
                bn = (b + 4) % NBUF
                if not first:
                    wait_scatter(bn)
                start_gather(j + 4, bn)

        for j in range(4):
            start_gather(j, j)
        step(0, 0, first=True)

        @pl.loop(0, 24)
        def _ring(t):
            j0 = 5 * t + 1
            for i in range(5):
                step(j0 + i, (1 + i) % NBUF)

        for j in range(NCHUNK - 4, NCHUNK):
            step(j, j % NBUF, last=True)

        for b in range(NBUF):
            wait_scatter(b)
        if with_counts:
            pltpu.make_async_copy(ones, cacc.at[dstb.at[0]], csem).wait()

        plsc.subcore_barrier()

        sl = pl.ds(s * RPT, RPT)
        pltpu.sync_copy(acc.at[sl], out_hbm.at[c, sl])
        if with_counts:
            pltpu.sync_copy(cacc.at[sl], cnt_hbm.at[c, sl])

    return pl.kernel(body, out_type=out_type, mesh=mesh,
                     scratch_types=scratch,
                     compiler_params=pltpu.CompilerParams(
                         use_tc_tiling_on_sc=False))


_segsum_counts = _segsum_kernel(True)
_segsum_plain = _segsum_kernel(False)


def _proj2_body(x_ref, wa_ref, wb_ref, pa_ref, pb_ref):
    xv = x_ref[...]
    pa_ref[...] = jnp.dot(xv, wa_ref[...], preferred_element_type=jnp.float32)
    pb_ref[...] = jnp.dot(xv, wb_ref[...], preferred_element_type=jnp.float32)


def _proj2(x, wa, wb):
    m = x.shape[0]
    h = wa.shape[1]
    return pl.pallas_call(
        _proj2_body,
        out_shape=[jax.ShapeDtypeStruct((m, h), jnp.float32)] * 2,
    )(x, wa, wb)


def _mid_body(accp_ref, cntp_ref, r_ref, b_ref, wl_ref, wr_ref,
              p2_ref, r2_ref):
    sacc = accp_ref[0] + accp_ref[1]
    cnt = cntp_ref[0, :, 0:1] + cntp_ref[1, :, 0:1]
    aggr = sacc / jnp.maximum(cnt, 1.0)
    out = aggr + b_ref[...][None, :] + r_ref[...]
    nrm = jnp.sqrt(jnp.sum(out * out, axis=-1, keepdims=True))
    h = jax.nn.relu(out / jnp.maximum(nrm, 1e-12))
    p2_ref[...] = jnp.dot(h, wl_ref[...], preferred_element_type=jnp.float32)
    r2_ref[...] = jnp.dot(h, wr_ref[...], preferred_element_type=jnp.float32)


def _final_body(accp_ref, cntp_ref, r_ref, b_ref, wlin_ref, blin_ref,
                o_ref):
    sacc = accp_ref[0] + accp_ref[1]
    cnt = cntp_ref[0, :, 0:1] + cntp_ref[1, :, 0:1]
    aggr = sacc / jnp.maximum(cnt, 1.0)
    out = aggr + b_ref[...][None, :] + r_ref[...]
    nrm = jnp.sqrt(jnp.sum(out * out, axis=-1, keepdims=True))
    h = jax.nn.relu(out / jnp.maximum(nrm, 1e-12))
    o_ref[...] = (jnp.dot(h, wlin_ref[...], preferred_element_type=jnp.float32)
                  + blin_ref[...][None, :])


def kernel(x, edge_index, edge_weight, W1l, b1l, W1r, W2l, b2l, W2r,
           Wlin, blin):
    src = edge_index[0].reshape(NW, NCHUNK, K)
    dst = edge_index[1].reshape(NW, NCHUNK, K)
    w3 = edge_weight.reshape(NW, NCHUNK, K)
    xp = jnp.pad(x, ((0, NPAD - N), (0, 0)))

    p1, r1 = _proj2(xp, W1l, W1r)
    acc1, cnt = _segsum_counts(p1, src, dst, w3)

    p2, r2 = pl.pallas_call(
        _mid_body,
        out_shape=[jax.ShapeDtypeStruct((NPAD, H), jnp.float32)] * 2,
    )(acc1, cnt, r1, b1l, W2l, W2r)

    acc2 = _segsum_plain(p2, src, dst, w3)

    out = pl.pallas_call(
        _final_body,
        out_shape=jax.ShapeDtypeStruct((NPAD, C), jnp.float32),
    )(acc2, cnt, r2, b2l, Wlin, blin)
    return out[:N]


# speedup vs baseline: 19.5906x; 1.0267x over previous
import functools

import jax
import jax.numpy as jnp
from jax import lax
from jax.experimental import pallas as pl
from jax.experimental.pallas import tpu as pltpu
from jax.experimental.pallas import tpu_sc as plsc

N = 10000
NPAD = 10240
E = 320000
D_IN = 128
H = 32
C = 2

NC = 2
NS = 16
NW = NC * NS
PER_W = E // NW
K = 80
NCHUNK = PER_W // K
CW = 16
RPT = NPAD // NS


def _segsum_kernel(with_counts):
    mesh = plsc.VectorSubcoreMesh(core_axis_name="c", subcore_axis_name="s")
    if with_counts:
        out_type = [jax.ShapeDtypeStruct((NC, NPAD, H), jnp.float32),
                    jax.ShapeDtypeStruct((NC, NPAD, CW), jnp.float32)]
    else:
        out_type = jax.ShapeDtypeStruct((NC, NPAD, H), jnp.float32)
    NBUF = 5
    scratch = (
        [pltpu.VMEM((PER_W,), jnp.int32),
         pltpu.VMEM((NCHUNK, K), jnp.int32),
         pltpu.VMEM((PER_W,), jnp.float32)]
        + [pltpu.VMEM((K, H), jnp.float32)] * NBUF
        + [pltpu.VMEM((RPT, H), jnp.float32),
           pltpu.VMEM_SHARED((NPAD, H), jnp.float32)]
        + [pltpu.SemaphoreType.DMA] * (2 * NBUF)
    )
    if with_counts:
        scratch += [
            pltpu.VMEM((K, CW), jnp.float32),
            pltpu.VMEM((RPT, CW), jnp.float32),
            pltpu.VMEM_SHARED((NPAD, CW), jnp.float32),
            pltpu.SemaphoreType.DMA,
        ]

    def body(p_hbm, src_hbm, dst_hbm, w_hbm, out_hbm, *rest):
        rest = list(rest)
        cnt_hbm = rest.pop(0) if with_counts else None
        srcb, dstb, wb = rest[0:3]
        bufs = rest[3:3 + NBUF]
        zbuf = rest[3 + NBUF]
        acc = rest[4 + NBUF]
        gsem = rest[5 + NBUF:5 + 2 * NBUF]
        ssem = rest[5 + 2 * NBUF:5 + 3 * NBUF]
        if with_counts:
            ones, zcnt, cacc, csem = rest[5 + 3 * NBUF:]
        c = lax.axis_index("c")
        s = lax.axis_index("s")
        wid = c * NS + s

        pltpu.async_copy(src_hbm.at[wid], srcb, gsem[0])
        pltpu.async_copy(dst_hbm.at[wid], dstb, gsem[0])
        pltpu.async_copy(w_hbm.at[wid], wb, gsem[0])

        @pl.loop(0, RPT)
        def _zero(i):
            for j in range(0, H, 16):
                zbuf[i, pl.ds(j, 16)] = jnp.zeros((16,), jnp.float32)
            if with_counts:
                zcnt[i, pl.ds(0, CW)] = jnp.zeros((CW,), jnp.float32)

        pltpu.sync_copy(zbuf, acc.at[pl.ds(s * RPT, RPT)])
        if with_counts:
            pltpu.sync_copy(zcnt, cacc.at[pl.ds(s * RPT, RPT)])

            @pl.loop(0, K)
            def _fill(i):
                ones[i, pl.ds(0, CW)] = jnp.ones((CW,), jnp.float32)

        pltpu.make_async_copy(src_hbm.at[wid], srcb, gsem[0]).wait()
        pltpu.make_async_copy(dst_hbm.at[wid], dstb, gsem[0]).wait()
        pltpu.make_async_copy(w_hbm.at[wid], wb, gsem[0]).wait()
        plsc.subcore_barrier()

        def start_gather(j, b):
            pltpu.async_copy(p_hbm.at[srcb.at[pl.ds(j * K, K)]],
                             bufs[b], gsem[b])

        def wait_gather(b):
            pltpu.make_async_copy(p_hbm.at[srcb.at[pl.ds(0, K)]], bufs[b],
                                  gsem[b]).wait()

        def wait_scatter(b):
            pltpu.make_async_copy(bufs[b], acc.at[dstb.at[0]],
                                  ssem[b]).wait()

        def scale(buf, j):
            @pl.loop(0, K, step=16)
            def _scale(i):
                wv16 = wb[pl.ds(j * K + i, 16)]
                for e in range(16):
                    wv = wv16[e]
                    for jj in range(0, H, 16):
                        buf[i + e, pl.ds(jj, 16)] = (
                            buf[i + e, pl.ds(jj, 16)] * wv)

        def step(j, b, first=False, last=False):
            wait_gather(b)
            scale(bufs[b], j)
            pltpu.async_copy(bufs[b], acc.at[dstb.at[j]], ssem[b], add=True)
            if with_counts:
                if not first:
                    pltpu.make_async_copy(ones, cacc.at[dstb.at[0]],
                                          csem).wait()
                pltpu.async_copy(ones, cacc.at[dstb.at[j]], csem, add=True)
            if not last:
                bn = (b + 4) % NBUF
                if not first:
                    wait_scatter(bn)
                start_gather(j + 4, bn)

        for j in range(4):
            start_gather(j, j)
        step(0, 0, first=True)

        @pl.loop(0, 24)
        def _ring(t):
            j0 = 5 * t + 1
            for i in range(5):
                step(j0 + i, (1 + i) % NBUF)

        for j in range(NCHUNK - 4, NCHUNK):
            step(j, j % NBUF, last=True)

        for b in range(NBUF):
            wait_scatter(b)
        if with_counts:
            pltpu.make_async_copy(ones, cacc.at[dstb.at[0]], csem).wait()

        plsc.subcore_barrier()

        sl = pl.ds(s * RPT, RPT)
        pltpu.sync_copy(acc.at[sl], out_hbm.at[c, sl])
        if with_counts:
            pltpu.sync_copy(cacc.at[sl], cnt_hbm.at[c, sl])

    return pl.kernel(body, out_type=out_type, mesh=mesh,
                     scratch_types=scratch,
                     compiler_params=pltpu.CompilerParams(
                         use_tc_tiling_on_sc=False))


_segsum_counts = _segsum_kernel(True)
_segsum_plain = _segsum_kernel(False)


def _proj2_body(x_ref, wa_ref, wb_ref, pa_ref, pb_ref):
    xv = x_ref[...]
    pa_ref[...] = jnp.dot(xv, wa_ref[...], preferred_element_type=jnp.float32)
    pb_ref[...] = jnp.dot(xv, wb_ref[...], preferred_element_type=jnp.float32)


def _proj2(x, wa, wb):
    m = x.shape[0]
    h = wa.shape[1]
    return pl.pallas_call(
        _proj2_body,
        out_shape=[jax.ShapeDtypeStruct((m, h), jnp.float32)] * 2,
    )(x, wa, wb)


def _mid_body(accp_ref, cntp_ref, r_ref, b_ref, wl_ref, wr_ref,
              p2_ref, r2_ref):
    sacc = accp_ref[0, :N] + accp_ref[1, :N]
    cnt = cntp_ref[0, :N, 0:1] + cntp_ref[1, :N, 0:1]
    aggr = sacc / jnp.maximum(cnt, 1.0)
    out = aggr + b_ref[...][None, :] + r_ref[...]
    nrm = jnp.sqrt(jnp.sum(out * out, axis=-1, keepdims=True))
    h = jax.nn.relu(out / jnp.maximum(nrm, 1e-12))
    p2_ref[...] = jnp.dot(h, wl_ref[...], preferred_element_type=jnp.float32)
    r2_ref[...] = jnp.dot(h, wr_ref[...], preferred_element_type=jnp.float32)


def _final_body(accp_ref, cntp_ref, r_ref, b_ref, wlin_ref, blin_ref,
                o_ref):
    sacc = accp_ref[0, :N] + accp_ref[1, :N]
    cnt = cntp_ref[0, :N, 0:1] + cntp_ref[1, :N, 0:1]
    aggr = sacc / jnp.maximum(cnt, 1.0)
    out = aggr + b_ref[...][None, :] + r_ref[...]
    nrm = jnp.sqrt(jnp.sum(out * out, axis=-1, keepdims=True))
    h = jax.nn.relu(out / jnp.maximum(nrm, 1e-12))
    o_ref[...] = (jnp.dot(h, wlin_ref[...], preferred_element_type=jnp.float32)
                  + blin_ref[...][None, :])


def kernel(x, edge_index, edge_weight, W1l, b1l, W1r, W2l, b2l, W2r,
           Wlin, blin):
    src = edge_index[0].reshape(NW, PER_W)
    dst = edge_index[1].reshape(NW, NCHUNK, K)
    w2 = edge_weight.reshape(NW, PER_W)

    p1, r1 = _proj2(x, W1l, W1r)
    acc1, cnt = _segsum_counts(p1, src, dst, w2)

    p2, r2 = pl.pallas_call(
        _mid_body,
        out_shape=[jax.ShapeDtypeStruct((N, H), jnp.float32)] * 2,
    )(acc1, cnt, r1, b1l, W2l, W2r)

    acc2 = _segsum_plain(p2, src, dst, w2)

    out = pl.pallas_call(
        _final_body,
        out_shape=jax.ShapeDtypeStruct((N, C), jnp.float32),
    )(acc2, cnt, r2, b2l, Wlin, blin)
    return out
